# baseline (speedup 1.0000x reference)
import functools

import jax
import jax.numpy as jnp
import numpy as np
from jax import lax
from jax.experimental import pallas as pl
from jax.experimental.pallas import tpu as pltpu
from jax.experimental.pallas import tpu_sc as plsc

N = 10000
E = 160000
NC = 2
NS = 16
NW = NC * NS
EPW = E // NW
CH = 128
NCH = 40
EPP = NCH * CH
PADW = EPP - EPW
EP = NW * EPP
RPT = 624
NTL = RPT * NS
TAIL = N - NTL

BE = 1280
GEP = EP // BE
BN = 2000
GN = N // BN


@functools.lru_cache(maxsize=None)
def _get_mesh():
  return plsc.VectorSubcoreMesh(
      core_axis_name="c", subcore_axis_name="s", num_cores=NC, num_subcores=NS)


_SC_PARAMS = dict(compiler_params=pltpu.CompilerParams(use_tc_tiling_on_sc=False))


def _wid():
  return lax.axis_index("s") * NC + lax.axis_index("c")


@functools.lru_cache(maxsize=None)
def _make_gather16():
  @functools.partial(
      pl.kernel, mesh=_get_mesh(), **_SC_PARAMS,
      out_type=jax.ShapeDtypeStruct((NW, EPP, 16), jnp.float32),
      scratch_types=[
          pltpu.VMEM((NCH, CH), jnp.int32),
          pltpu.VMEM((EPP, 16), jnp.float32),
          pltpu.SemaphoreType.DMA,
      ],
      name="sc_gather16",
  )
  def gk(table_hbm, idx_hbm, out_hbm, idx_v, rows_v, sem):
    w = _wid()
    pltpu.sync_copy(idx_hbm.at[w], idx_v)

    def fire(j, c):
      pltpu.async_copy(table_hbm.at[idx_v.at[j]],
                       rows_v.at[pl.ds(j * CH, CH)], sem)
      return c

    lax.fori_loop(0, NCH, fire, 0)

    def drain(j, c):
      pltpu.make_async_copy(table_hbm.at[idx_v.at[0]],
                            rows_v.at[pl.ds(0, CH)], sem).wait()
      return c

    lax.fori_loop(0, NCH, drain, 0)
    pltpu.sync_copy(rows_v, out_hbm.at[w])

  return gk


@functools.lru_cache(maxsize=None)
def _make_gather16x2():
  @functools.partial(
      pl.kernel, mesh=_get_mesh(), **_SC_PARAMS,
      out_type=(jax.ShapeDtypeStruct((NW, EPP, 16), jnp.float32),
                jax.ShapeDtypeStruct((NW, EPP, 16), jnp.float32)),
      scratch_types=[
          pltpu.VMEM((NCH, CH), jnp.int32),
          pltpu.VMEM((NCH, CH), jnp.int32),
          pltpu.VMEM((EPP, 16), jnp.float32),
          pltpu.SemaphoreType.DMA,
      ],
      name="sc_gather16x2",
  )
  def gk(table_hbm, idxa_hbm, idxb_hbm, outa_hbm, outb_hbm,
         idxa_v, idxb_v, rows_v, sem):
    w = _wid()
    pltpu.sync_copy(idxa_hbm.at[w], idxa_v)
    pltpu.sync_copy(idxb_hbm.at[w], idxb_v)
    for idx_v, out_hbm in ((idxa_v, outa_hbm), (idxb_v, outb_hbm)):

      def fire(j, c, _iv=idx_v):
        pltpu.async_copy(table_hbm.at[_iv.at[j]],
                         rows_v.at[pl.ds(j * CH, CH)], sem)
        return c

      lax.fori_loop(0, NCH, fire, 0)

      def drain(j, c, _iv=idx_v):
        pltpu.make_async_copy(table_hbm.at[_iv.at[0]],
                              rows_v.at[pl.ds(0, CH)], sem).wait()
        return c

      lax.fori_loop(0, NCH, drain, 0)
      pltpu.sync_copy(rows_v, out_hbm.at[w])

  return gk


@functools.lru_cache(maxsize=None)
def _make_gather128():
  NB = 4
  NG = NCH // NB - 1

  @functools.partial(
      pl.kernel, mesh=_get_mesh(), **_SC_PARAMS,
      out_type=jax.ShapeDtypeStruct((NW, EPP, 128), jnp.float32),
      scratch_types=[
          pltpu.VMEM((NCH, CH), jnp.int32),
          pltpu.VMEM((CH, 128), jnp.float32),
          pltpu.VMEM((CH, 128), jnp.float32),
          pltpu.VMEM((CH, 128), jnp.float32),
          pltpu.VMEM((CH, 128), jnp.float32),
          pltpu.SemaphoreType.DMA,
          pltpu.SemaphoreType.DMA,
          pltpu.SemaphoreType.DMA,
          pltpu.SemaphoreType.DMA,
      ],
      name="sc_gather128",
  )
  def gk(table_hbm, idx_hbm, out_hbm, idx_v, r0, r1, r2, r3, s0, s1, s2, s3):
    w = _wid()
    bufs = (r0, r1, r2, r3)
    sems = (s0, s1, s2, s3)
    pltpu.sync_copy(idx_hbm.at[w], idx_v)

    for b in range(NB):
      pltpu.async_copy(table_hbm.at[idx_v.at[b]], bufs[b], sems[b])

    def group(g, c):
      for b in range(NB):
        j = g * NB + b
        pltpu.make_async_copy(table_hbm.at[idx_v.at[0]], bufs[b],
                              sems[b]).wait()
        pltpu.sync_copy(bufs[b], out_hbm.at[w, pl.ds(j * CH, CH)])
        pltpu.async_copy(table_hbm.at[idx_v.at[j + NB]], bufs[b], sems[b])
      return c

    lax.fori_loop(0, NG, group, 0)

    for b in range(NB):
      j = NG * NB + b
      pltpu.make_async_copy(table_hbm.at[idx_v.at[0]], bufs[b], sems[b]).wait()
      pltpu.sync_copy(bufs[b], out_hbm.at[w, pl.ds(j * CH, CH)])

  return gk


@functools.lru_cache(maxsize=None)
def _make_scatter(DV, with_cnt):
  STG = EPP if DV == 16 else EPP // 2
  NPH = EPP // STG
  CPP = NCH // NPH
  out_type = [jax.ShapeDtypeStruct((NC, N, DV), jnp.float32)]
  scratch = [
      pltpu.VMEM((NCH, CH), jnp.int32),
      pltpu.VMEM((STG, DV), jnp.float32),
      pltpu.VMEM_SHARED((N + 8, DV), jnp.float32),
      pltpu.SemaphoreType.DMA,
  ]
  if with_cnt:
    out_type.append(jax.ShapeDtypeStruct((NC, N, 16), jnp.float32))
    scratch += [
        pltpu.VMEM((CH, 16), jnp.float32),
        pltpu.VMEM_SHARED((N + 8, 16), jnp.float32),
        pltpu.SemaphoreType.DMA,
    ]

  @functools.partial(
      pl.kernel, mesh=_get_mesh(),
      out_type=tuple(out_type) if with_cnt else out_type[0],
      **_SC_PARAMS,
      scratch_types=scratch, name=f"sc_scatter_{DV}_{int(with_cnt)}",
  )
  def sk(msg_hbm, dst_hbm, zeros_hbm, ones_hbm, *rest):
    if with_cnt:
      agg_out, cnt_out, idx_v, rows_v, table_sh, sem, ones_v, cnt_sh, semc = rest
    else:
      agg_out, idx_v, rows_v, table_sh, sem = rest
    cid = lax.axis_index("c")
    sid = lax.axis_index("s")
    w = sid * NC + cid
    r0 = sid * RPT

    pltpu.sync_copy(zeros_hbm.at[pl.ds(r0, RPT)], table_sh.at[pl.ds(r0, RPT)])
    if with_cnt:
      pltpu.sync_copy(zeros_hbm.at[pl.ds(r0, RPT), :16], cnt_sh.at[pl.ds(r0, RPT)])
      pltpu.sync_copy(ones_hbm, ones_v)

    @pl.when(sid == 0)
    def _zero_tail():
      pltpu.sync_copy(zeros_hbm.at[pl.ds(NTL, TAIL)], table_sh.at[pl.ds(NTL, TAIL)])
      if with_cnt:
        pltpu.sync_copy(zeros_hbm.at[pl.ds(NTL, TAIL), :16],
                        cnt_sh.at[pl.ds(NTL, TAIL)])

    pltpu.sync_copy(dst_hbm.at[w], idx_v)
    plsc.subcore_barrier()

    for p in range(NPH):
      pltpu.sync_copy(msg_hbm.at[w, pl.ds(p * STG, STG)], rows_v)

      def fire(j, c, _p=p):
        pltpu.async_copy(rows_v.at[pl.ds(j * CH, CH)],
                         table_sh.at[idx_v.at[_p * CPP + j]], sem, add=True)
        if with_cnt:
          pltpu.async_copy(ones_v, cnt_sh.at[idx_v.at[_p * CPP + j]],
                           semc, add=True)
        return c

      lax.fori_loop(0, CPP, fire, 0)

      def drain(j, c):
        pltpu.make_async_copy(msg_hbm.at[w, pl.ds(0, CH)],
                              rows_v.at[pl.ds(0, CH)], sem).wait()
        if with_cnt:
          pltpu.make_async_copy(msg_hbm.at[w, pl.ds(0, CH), :16],
                                ones_v, semc).wait()
        return c

      lax.fori_loop(0, CPP, drain, 0)

    plsc.subcore_barrier()

    pltpu.sync_copy(table_sh.at[pl.ds(r0, RPT)], agg_out.at[cid, pl.ds(r0, RPT)])
    if with_cnt:
      pltpu.sync_copy(cnt_sh.at[pl.ds(r0, RPT)], cnt_out.at[cid, pl.ds(r0, RPT)])

    @pl.when(sid == 0)
    def _write_tail():
      pltpu.sync_copy(table_sh.at[pl.ds(NTL, TAIL)],
                      agg_out.at[cid, pl.ds(NTL, TAIL)])
      if with_cnt:
        pltpu.sync_copy(cnt_sh.at[pl.ds(NTL, TAIL)],
                        cnt_out.at[cid, pl.ds(NTL, TAIL)])

  return sk


def _sc_gather(table, idx3, d):
  if d == 128:
    return _make_gather128()(table, idx3)
  return _make_gather16()(table, idx3)


def _sc_gather2(table, idxa3, idxb3):
  return _make_gather16x2()(table, idxa3, idxb3)


def _sc_scatter(msg3, dst3, zeros, ones, dv, with_cnt):
  return _make_scatter(dv, with_cnt)(msg3, dst3, zeros, ones)


def _msg_body(xg_ref, ea_ref, w1_ref, b1_ref, w2p_ref, r_ref, s_ref,
              b2r_ref, o_ref):
  xg = xg_ref[...].astype(jnp.bfloat16)
  h = jnp.maximum(
      jnp.dot(ea_ref[...], w1_ref[...], preferred_element_type=jnp.float32)
      + b1_ref[...], 0.0)
  v = jnp.dot(xg, w2p_ref[...], preferred_element_type=jnp.float32)
  he = jnp.dot(h.astype(jnp.bfloat16), r_ref[...],
               preferred_element_type=jnp.float32)
  o_ref[...] = (
      jnp.dot((v * he).astype(jnp.bfloat16), s_ref[...],
              preferred_element_type=jnp.float32)
      + jnp.dot(xg, b2r_ref[...], preferred_element_type=jnp.float32))


def _run_msg(xg, ea_p, w1, b1, w2p, r_mat, s_mat, b2r, outd, ind):
  w1 = w1.astype(jnp.bfloat16)
  w2p = w2p.astype(jnp.bfloat16)
  b2r = b2r.astype(jnp.bfloat16)
  r_mat = r_mat.astype(jnp.bfloat16)
  s_mat = s_mat.astype(jnp.bfloat16)
  return pl.pallas_call(
      _msg_body,
      grid=(GEP,),
      in_specs=[
          pl.BlockSpec((BE, ind), lambda i: (i, 0)),
          pl.BlockSpec((BE, 16), lambda i: (i, 0)),
          pl.BlockSpec(w1.shape, lambda i: (0, 0)),
          pl.BlockSpec(b1.shape, lambda i: (0, 0)),
          pl.BlockSpec(w2p.shape, lambda i: (0, 0)),
          pl.BlockSpec(r_mat.shape, lambda i: (0, 0)),
          pl.BlockSpec(s_mat.shape, lambda i: (0, 0)),
          pl.BlockSpec(b2r.shape, lambda i: (0, 0)),
      ],
      out_specs=pl.BlockSpec((BE, outd), lambda i: (i, 0)),
      out_shape=jax.ShapeDtypeStruct((EP, outd), jnp.float32),
  )(xg, ea_p, w1, b1, w2p, r_mat, s_mat, b2r)


def _msg_body_packed(xgp_ref, eap_ref, w18_ref, b18_ref, w8_ref, r8_ref,
                     s8_ref, b8_ref, o_ref):
  xgp = xgp_ref[...].astype(jnp.bfloat16)
  h8 = jnp.maximum(
      jnp.dot(eap_ref[...].astype(jnp.bfloat16), w18_ref[...],
              preferred_element_type=jnp.float32) + b18_ref[...], 0.0)
  v8 = jnp.dot(xgp, w8_ref[...], preferred_element_type=jnp.float32)
  he8 = jnp.dot(h8.astype(jnp.bfloat16), r8_ref[...],
                preferred_element_type=jnp.float32)
  o_ref[...] = (
      jnp.dot((v8 * he8).astype(jnp.bfloat16), s8_ref[...],
              preferred_element_type=jnp.float32)
      + jnp.dot(xgp, b8_ref[...], preferred_element_type=jnp.float32))


def _run_msg_packed(xgp, eap8, w1, b1, w2p, b2r):
  BE8 = BE // 8
  w18 = jnp.kron(jnp.eye(8, dtype=jnp.float32), w1).astype(jnp.bfloat16)
  b18 = jnp.tile(b1.reshape(1, -1), (1, 8))
  w8 = jnp.kron(jnp.eye(8, dtype=jnp.float32), w2p).astype(jnp.bfloat16)
  r8 = jnp.asarray(np.kron(np.eye(8, dtype=np.float32), _R32)
                   ).astype(jnp.bfloat16)
  s8 = jnp.asarray(np.kron(np.eye(8, dtype=np.float32), _S32)
                   ).astype(jnp.bfloat16)
  b8 = jnp.kron(jnp.eye(8, dtype=jnp.float32), b2r).astype(jnp.bfloat16)
  return pl.pallas_call(
      _msg_body_packed,
      grid=(GEP,),
      in_specs=[
          pl.BlockSpec((BE8, 128), lambda i: (i, 0)),
          pl.BlockSpec((BE8, 128), lambda i: (i, 0)),
          pl.BlockSpec((128, 256), lambda i: (0, 0)),
          pl.BlockSpec((1, 256), lambda i: (0, 0)),
          pl.BlockSpec((128, 4096), lambda i: (0, 0)),
          pl.BlockSpec((256, 4096), lambda i: (0, 0)),
          pl.BlockSpec((4096, 128), lambda i: (0, 0)),
          pl.BlockSpec((128, 128), lambda i: (0, 0)),
      ],
      out_specs=pl.BlockSpec((BE8, 128), lambda i: (i, 0)),
      out_shape=jax.ShapeDtypeStruct((EP // 8, 128), jnp.float32),
  )(xgp, eap8, w18, b18, w8, r8, s8, b8)


def _node_body(aggp_ref, cntp_ref, f_ref, root_ref, bias_ref, o_ref):
  agg = aggp_ref[0] + aggp_ref[1]
  cnt = cntp_ref[0] + cntp_ref[1]
  o_ref[...] = jnp.maximum(
      agg / jnp.maximum(cnt, 1.0)
      + jnp.dot(f_ref[...], root_ref[...], preferred_element_type=jnp.float32)
      + bias_ref[...], 0.0)


def _run_node(aggp, cntp, f, root, bias, ind):
  return pl.pallas_call(
      _node_body,
      grid=(GN,),
      in_specs=[
          pl.BlockSpec((NC, BN, 16), lambda i: (0, i, 0)),
          pl.BlockSpec((NC, BN, 16), lambda i: (0, i, 0)),
          pl.BlockSpec((BN, ind), lambda i: (i, 0)),
          pl.BlockSpec((ind, 16), lambda i: (0, 0)),
          pl.BlockSpec((1, 16), lambda i: (0, 0)),
      ],
      out_specs=pl.BlockSpec((BN, 16), lambda i: (i, 0)),
      out_shape=jax.ShapeDtypeStruct((N, 16), jnp.float32),
  )(aggp, cntp, f, root, bias)


def _mulv_body(aggp_ref, h2_ref, mur_ref, lvr_ref, bmu_ref, blv_ref, eps_ref,
               mu_ref, lv_ref, z_ref):
  aggmu = aggp_ref[0, :, :16] + aggp_ref[1, :, :16]
  agglv = aggp_ref[0, :, 16:] + aggp_ref[1, :, 16:]
  mu = aggmu + jnp.dot(h2_ref[...], mur_ref[...],
                       preferred_element_type=jnp.float32) + bmu_ref[...]
  lv = agglv + jnp.dot(h2_ref[...], lvr_ref[...],
                       preferred_element_type=jnp.float32) + blv_ref[...]
  mu_ref[...] = mu
  lv_ref[...] = lv
  lvc = jnp.clip(lv, -5.0, 5.0)
  z_ref[...] = mu + eps_ref[...] * jnp.exp(0.5 * lvc)


_S8 = np.kron(np.eye(8, dtype=np.float32), np.ones((16, 1), np.float32))


def _dec_body(zs_ref, zd_ref, s8_ref, o_ref):
  s = jnp.dot(zs_ref[...] * zd_ref[...], s8_ref[...],
              preferred_element_type=jnp.float32)
  o_ref[...] = 1.0 / (1.0 + jnp.exp(-s))


def _pool_body(batch_ref, z_ref, sums_ref, cnt_ref):
  i = pl.program_id(0)
  b = batch_ref[0, 0, :]
  oh = (b[:, None] == lax.broadcasted_iota(jnp.int32, (BN, 64), 1)
        ).astype(jnp.float32)
  ps = lax.dot_general(oh, z_ref[...], (((0,), (0,)), ((), ())),
                       preferred_element_type=jnp.float32)
  pc = lax.dot_general(oh, jnp.ones((BN, 16), jnp.float32),
                       (((0,), (0,)), ((), ())),
                       preferred_element_type=jnp.float32)

  @pl.when(i == 0)
  def _():
    sums_ref[...] = jnp.zeros_like(sums_ref)
    cnt_ref[...] = jnp.zeros_like(cnt_ref)

  sums_ref[...] += ps
  cnt_ref[...] += pc


def _cls_body(sums_ref, cnt_ref, w1_ref, b1_ref, w2_ref, b2_ref, o_ref):
  gemb = sums_ref[...] / jnp.maximum(cnt_ref[...], 1.0)
  a = jnp.maximum(
      jnp.dot(gemb, w1_ref[...], preferred_element_type=jnp.float32)
      + b1_ref[...], 0.0)
  o_ref[...] = jnp.dot(a, w2_ref[...], preferred_element_type=jnp.float32) \
      + b2_ref[...]


def _prep_w2(w2, b2, ind, outd):
  w2p = w2.reshape(32, ind, outd).transpose(1, 0, 2).reshape(ind, 32 * outd)
  return w2p, b2.reshape(ind, outd)


_S32 = np.kron(np.ones((32, 1), np.float32), np.eye(16, dtype=np.float32))
_S64 = np.kron(np.eye(2, dtype=np.float32), _S32)


_R32 = np.kron(np.eye(32, dtype=np.float32), np.ones((1, 16), np.float32))
_R64 = np.kron(np.eye(64, dtype=np.float32), np.ones((1, 16), np.float32))


def _pad_edges(a, fill_shape_tail, fill):
  a3 = a.reshape((NW, EPW) + fill_shape_tail)
  pad = jnp.full((NW, PADW) + fill_shape_tail, fill, a.dtype)
  return jnp.concatenate([a3, pad], axis=1)


def kernel(x, edge_index, edge_attr, batch, c1_w1, c1_b1, c1_w2, c1_b2,
           c1_root, c1_bias, c2_w1, c2_b1, c2_w2, c2_b2, c2_root, c2_bias,
           cmu_w1, cmu_b1, cmu_w2, cmu_b2, cmu_root, cmu_bias, clv_w1,
           clv_b1, clv_w2, clv_b2, clv_root, clv_bias, cls_w1, cls_b1,
           cls_w2, cls_b2):
  src3 = _pad_edges(edge_index[0], (), 0).reshape(NW, NCH, CH)
  dst3 = _pad_edges(edge_index[1], (), N).reshape(NW, NCH, CH)
  ea_p = _pad_edges(edge_attr, (16,), 0).reshape(EP, 16).astype(jnp.bfloat16)
  ea_flat = jnp.concatenate(
      [edge_attr.reshape(NW, EPW * 16),
       jnp.zeros((NW, PADW * 16), jnp.float32)], axis=1)
  eap8 = ea_flat.reshape(EP // 8, 128)
  zeros16 = jnp.zeros((N, 16), jnp.float32)
  zeros32 = jnp.zeros((N, 32), jnp.float32)
  ones_rows = jnp.ones((CH, 16), jnp.float32)

  w2p1, b2r1 = _prep_w2(c1_w2, c1_b2, 128, 16)
  xg1 = _sc_gather(x, src3, 128).reshape(EP, 128)
  msg1 = _run_msg(xg1, ea_p, c1_w1, c1_b1.reshape(1, 32), w2p1,
                  jnp.asarray(_R32), jnp.asarray(_S32), b2r1, 16, 128)
  agg1p, cnt1p = _sc_scatter(msg1.reshape(NW, EPP, 16), dst3, zeros16,
                             ones_rows, 16, True)
  h1 = _run_node(agg1p, cnt1p, x, c1_root, c1_bias.reshape(1, 16), 128)

  w2p2, b2r2 = _prep_w2(c2_w2, c2_b2, 16, 16)
  xg2p = _sc_gather(h1, src3, 16).reshape(EP // 8, 128)
  msg2 = _run_msg_packed(xg2p, eap8, c2_w1, c2_b1, w2p2, b2r2)
  agg2p = _sc_scatter(msg2.reshape(NW, EPP, 16), dst3, zeros16,
                      ones_rows, 16, False)
  h2 = _run_node(agg2p, cnt1p, h1, c2_root, c2_bias.reshape(1, 16), 16)

  w2pm, b2rm = _prep_w2(cmu_w2, cmu_b2, 16, 16)
  w2pl, b2rl = _prep_w2(clv_w2, clv_b2, 16, 16)
  w2pc = jnp.concatenate([w2pm, w2pl], axis=1)
  b2rc = jnp.concatenate([b2rm, b2rl], axis=1)
  w1ml = jnp.concatenate([cmu_w1, clv_w1], axis=1)
  b1ml = jnp.concatenate([cmu_b1, clv_b1]).reshape(1, 64)
  xg3 = _sc_gather(h2, src3, 16).reshape(EP, 16)
  msg3 = _run_msg(xg3, ea_p, w1ml, b1ml, w2pc,
                  jnp.asarray(_R64), jnp.asarray(_S64), b2rc, 32, 16)
  agg3p = _sc_scatter(msg3.reshape(NW, EPP, 32), dst3, zeros32,
                      ones_rows, 32, False)

  eps = jax.random.normal(jax.random.key(42), (N, 16), dtype=jnp.float32)
  mu, logvar, z = pl.pallas_call(
      _mulv_body,
      grid=(GN,),
      in_specs=[
          pl.BlockSpec((NC, BN, 32), lambda i: (0, i, 0)),
          pl.BlockSpec((BN, 16), lambda i: (i, 0)),
          pl.BlockSpec((16, 16), lambda i: (0, 0)),
          pl.BlockSpec((16, 16), lambda i: (0, 0)),
          pl.BlockSpec((1, 16), lambda i: (0, 0)),
          pl.BlockSpec((1, 16), lambda i: (0, 0)),
          pl.BlockSpec((BN, 16), lambda i: (i, 0)),
      ],
      out_specs=[
          pl.BlockSpec((BN, 16), lambda i: (i, 0)),
          pl.BlockSpec((BN, 16), lambda i: (i, 0)),
          pl.BlockSpec((BN, 16), lambda i: (i, 0)),
      ],
      out_shape=[
          jax.ShapeDtypeStruct((N, 16), jnp.float32),
          jax.ShapeDtypeStruct((N, 16), jnp.float32),
          jax.ShapeDtypeStruct((N, 16), jnp.float32),
      ],
  )(agg3p, h2, cmu_root, clv_root, cmu_bias.reshape(1, 16),
    clv_bias.reshape(1, 16), eps)

  zs, zd = _sc_gather2(z, src3, dst3)
  BE8 = BE // 8
  adj_pred = pl.pallas_call(
      _dec_body,
      grid=(GEP,),
      in_specs=[
          pl.BlockSpec((BE8, 128), lambda i: (i, 0)),
          pl.BlockSpec((BE8, 128), lambda i: (i, 0)),
          pl.BlockSpec((128, 8), lambda i: (0, 0)),
      ],
      out_specs=pl.BlockSpec((BE8, 8), lambda i: (i, 0)),
      out_shape=jax.ShapeDtypeStruct((EP // 8, 8), jnp.float32),
  )(zs.reshape(EP // 8, 128), zd.reshape(EP // 8, 128), jnp.asarray(_S8))
  adj_pred = adj_pred.reshape(NW, EPP)[:, :EPW].reshape(E)

  sums, cnt = pl.pallas_call(
      _pool_body,
      grid=(GN,),
      in_specs=[
          pl.BlockSpec((1, 1, BN), lambda i: (i, 0, 0)),
          pl.BlockSpec((BN, 16), lambda i: (i, 0)),
      ],
      out_specs=[
          pl.BlockSpec((64, 16), lambda i: (0, 0)),
          pl.BlockSpec((64, 16), lambda i: (0, 0)),
      ],
      out_shape=[
          jax.ShapeDtypeStruct((64, 16), jnp.float32),
          jax.ShapeDtypeStruct((64, 16), jnp.float32),
      ],
  )(batch.reshape(GN, 1, BN), z)
  class_logits = pl.pallas_call(
      _cls_body,
      in_specs=[
          pl.BlockSpec((64, 16), lambda: (0, 0)),
          pl.BlockSpec((64, 16), lambda: (0, 0)),
          pl.BlockSpec((16, 64), lambda: (0, 0)),
          pl.BlockSpec((1, 64), lambda: (0, 0)),
          pl.BlockSpec((64, 6), lambda: (0, 0)),
          pl.BlockSpec((1, 6), lambda: (0, 0)),
      ],
      out_specs=pl.BlockSpec((64, 6), lambda: (0, 0)),
      out_shape=jax.ShapeDtypeStruct((64, 6), jnp.float32),
  )(sums, cnt, cls_w1, cls_b1.reshape(1, 64), cls_w2, cls_b2.reshape(1, 6))

  return (adj_pred, mu, logvar, class_logits, z)


# speedup vs baseline: 1.0643x; 1.0643x over previous
import functools

import jax
import jax.numpy as jnp
import numpy as np
from jax import lax
from jax.experimental import pallas as pl
from jax.experimental.pallas import tpu as pltpu
from jax.experimental.pallas import tpu_sc as plsc

N = 10000
E = 160000
NC = 2
NS = 16
NW = NC * NS
EPW = E // NW
CH = 128
NCH = 40
EPP = NCH * CH
PADW = EPP - EPW
EP = NW * EPP
RPT = 624
NTL = RPT * NS
TAIL = N - NTL

BE = 5120
GEP = EP // BE
BN = 2000
GN = N // BN


@functools.lru_cache(maxsize=None)
def _get_mesh():
  return plsc.VectorSubcoreMesh(
      core_axis_name="c", subcore_axis_name="s", num_cores=NC, num_subcores=NS)


_SC_PARAMS = dict(compiler_params=pltpu.CompilerParams(use_tc_tiling_on_sc=False))


def _wid():
  return lax.axis_index("s") * NC + lax.axis_index("c")


@functools.lru_cache(maxsize=None)
def _make_gather16():
  @functools.partial(
      pl.kernel, mesh=_get_mesh(), **_SC_PARAMS,
      out_type=jax.ShapeDtypeStruct((NW, EPP, 16), jnp.float32),
      scratch_types=[
          pltpu.VMEM((NCH, CH), jnp.int32),
          pltpu.VMEM((EPP, 16), jnp.float32),
          pltpu.SemaphoreType.DMA,
      ],
      name="sc_gather16",
  )
  def gk(table_hbm, idx_hbm, out_hbm, idx_v, rows_v, sem):
    w = _wid()
    pltpu.sync_copy(idx_hbm.at[w], idx_v)

    def fire(j, c):
      pltpu.async_copy(table_hbm.at[idx_v.at[j]],
                       rows_v.at[pl.ds(j * CH, CH)], sem)
      return c

    lax.fori_loop(0, NCH, fire, 0)

    def drain(j, c):
      pltpu.make_async_copy(table_hbm.at[idx_v.at[0]],
                            rows_v.at[pl.ds(0, CH)], sem).wait()
      return c

    lax.fori_loop(0, NCH, drain, 0)
    pltpu.sync_copy(rows_v, out_hbm.at[w])

  return gk


@functools.lru_cache(maxsize=None)
def _make_gather16x2():
  @functools.partial(
      pl.kernel, mesh=_get_mesh(), **_SC_PARAMS,
      out_type=(jax.ShapeDtypeStruct((NW, EPP, 16), jnp.float32),
                jax.ShapeDtypeStruct((NW, EPP, 16), jnp.float32)),
      scratch_types=[
          pltpu.VMEM((NCH, CH), jnp.int32),
          pltpu.VMEM((NCH, CH), jnp.int32),
          pltpu.VMEM((EPP, 16), jnp.float32),
          pltpu.SemaphoreType.DMA,
      ],
      name="sc_gather16x2",
  )
  def gk(table_hbm, idxa_hbm, idxb_hbm, outa_hbm, outb_hbm,
         idxa_v, idxb_v, rows_v, sem):
    w = _wid()
    pltpu.sync_copy(idxa_hbm.at[w], idxa_v)
    pltpu.sync_copy(idxb_hbm.at[w], idxb_v)
    for idx_v, out_hbm in ((idxa_v, outa_hbm), (idxb_v, outb_hbm)):

      def fire(j, c, _iv=idx_v):
        pltpu.async_copy(table_hbm.at[_iv.at[j]],
                         rows_v.at[pl.ds(j * CH, CH)], sem)
        return c

      lax.fori_loop(0, NCH, fire, 0)

      def drain(j, c, _iv=idx_v):
        pltpu.make_async_copy(table_hbm.at[_iv.at[0]],
                              rows_v.at[pl.ds(0, CH)], sem).wait()
        return c

      lax.fori_loop(0, NCH, drain, 0)
      pltpu.sync_copy(rows_v, out_hbm.at[w])

  return gk


@functools.lru_cache(maxsize=None)
def _make_gather128():
  NB = 4
  NG = NCH // NB - 1

  @functools.partial(
      pl.kernel, mesh=_get_mesh(), **_SC_PARAMS,
      out_type=jax.ShapeDtypeStruct((NW, EPP, 128), jnp.float32),
      scratch_types=[
          pltpu.VMEM((NCH, CH), jnp.int32),
          pltpu.VMEM((CH, 128), jnp.float32),
          pltpu.VMEM((CH, 128), jnp.float32),
          pltpu.VMEM((CH, 128), jnp.float32),
          pltpu.VMEM((CH, 128), jnp.float32),
          pltpu.SemaphoreType.DMA,
          pltpu.SemaphoreType.DMA,
          pltpu.SemaphoreType.DMA,
          pltpu.SemaphoreType.DMA,
      ],
      name="sc_gather128",
  )
  def gk(table_hbm, idx_hbm, out_hbm, idx_v, r0, r1, r2, r3, s0, s1, s2, s3):
    w = _wid()
    bufs = (r0, r1, r2, r3)
    sems = (s0, s1, s2, s3)
    pltpu.sync_copy(idx_hbm.at[w], idx_v)

    for b in range(NB):
      pltpu.async_copy(table_hbm.at[idx_v.at[b]], bufs[b], sems[b])

    def group(g, c):
      for b in range(NB):
        j = g * NB + b
        pltpu.make_async_copy(table_hbm.at[idx_v.at[0]], bufs[b],
                              sems[b]).wait()
        pltpu.sync_copy(bufs[b], out_hbm.at[w, pl.ds(j * CH, CH)])
        pltpu.async_copy(table_hbm.at[idx_v.at[j + NB]], bufs[b], sems[b])
      return c

    lax.fori_loop(0, NG, group, 0)

    for b in range(NB):
      j = NG * NB + b
      pltpu.make_async_copy(table_hbm.at[idx_v.at[0]], bufs[b], sems[b]).wait()
      pltpu.sync_copy(bufs[b], out_hbm.at[w, pl.ds(j * CH, CH)])

  return gk


@functools.lru_cache(maxsize=None)
def _make_scatter(DV, with_cnt):
  STG = EPP if DV == 16 else EPP // 2
  NPH = EPP // STG
  CPP = NCH // NPH
  out_type = [jax.ShapeDtypeStruct((NC, N, DV), jnp.float32)]
  scratch = [
      pltpu.VMEM((NCH, CH), jnp.int32),
      pltpu.VMEM((STG, DV), jnp.float32),
      pltpu.VMEM_SHARED((N + 8, DV), jnp.float32),
      pltpu.SemaphoreType.DMA,
  ]
  if with_cnt:
    out_type.append(jax.ShapeDtypeStruct((NC, N, 16), jnp.float32))
    scratch += [
        pltpu.VMEM((CH, 16), jnp.float32),
        pltpu.VMEM_SHARED((N + 8, 16), jnp.float32),
        pltpu.SemaphoreType.DMA,
    ]

  @functools.partial(
      pl.kernel, mesh=_get_mesh(),
      out_type=tuple(out_type) if with_cnt else out_type[0],
      **_SC_PARAMS,
      scratch_types=scratch, name=f"sc_scatter_{DV}_{int(with_cnt)}",
  )
  def sk(msg_hbm, dst_hbm, zeros_hbm, ones_hbm, *rest):
    if with_cnt:
      agg_out, cnt_out, idx_v, rows_v, table_sh, sem, ones_v, cnt_sh, semc = rest
    else:
      agg_out, idx_v, rows_v, table_sh, sem = rest
    cid = lax.axis_index("c")
    sid = lax.axis_index("s")
    w = sid * NC + cid
    r0 = sid * RPT

    pltpu.sync_copy(zeros_hbm.at[pl.ds(r0, RPT)], table_sh.at[pl.ds(r0, RPT)])
    if with_cnt:
      pltpu.sync_copy(zeros_hbm.at[pl.ds(r0, RPT), :16], cnt_sh.at[pl.ds(r0, RPT)])
      pltpu.sync_copy(ones_hbm, ones_v)

    @pl.when(sid == 0)
    def _zero_tail():
      pltpu.sync_copy(zeros_hbm.at[pl.ds(NTL, TAIL)], table_sh.at[pl.ds(NTL, TAIL)])
      if with_cnt:
        pltpu.sync_copy(zeros_hbm.at[pl.ds(NTL, TAIL), :16],
                        cnt_sh.at[pl.ds(NTL, TAIL)])

    pltpu.sync_copy(dst_hbm.at[w], idx_v)
    plsc.subcore_barrier()

    for p in range(NPH):
      pltpu.sync_copy(msg_hbm.at[w, pl.ds(p * STG, STG)], rows_v)

      def fire(j, c, _p=p):
        pltpu.async_copy(rows_v.at[pl.ds(j * CH, CH)],
                         table_sh.at[idx_v.at[_p * CPP + j]], sem, add=True)
        if with_cnt:
          pltpu.async_copy(ones_v, cnt_sh.at[idx_v.at[_p * CPP + j]],
                           semc, add=True)
        return c

      lax.fori_loop(0, CPP, fire, 0)

      def drain(j, c):
        pltpu.make_async_copy(msg_hbm.at[w, pl.ds(0, CH)],
                              rows_v.at[pl.ds(0, CH)], sem).wait()
        if with_cnt:
          pltpu.make_async_copy(msg_hbm.at[w, pl.ds(0, CH), :16],
                                ones_v, semc).wait()
        return c

      lax.fori_loop(0, CPP, drain, 0)

    plsc.subcore_barrier()

    pltpu.sync_copy(table_sh.at[pl.ds(r0, RPT)], agg_out.at[cid, pl.ds(r0, RPT)])
    if with_cnt:
      pltpu.sync_copy(cnt_sh.at[pl.ds(r0, RPT)], cnt_out.at[cid, pl.ds(r0, RPT)])

    @pl.when(sid == 0)
    def _write_tail():
      pltpu.sync_copy(table_sh.at[pl.ds(NTL, TAIL)],
                      agg_out.at[cid, pl.ds(NTL, TAIL)])
      if with_cnt:
        pltpu.sync_copy(cnt_sh.at[pl.ds(NTL, TAIL)],
                        cnt_out.at[cid, pl.ds(NTL, TAIL)])

  return sk


def _sc_gather(table, idx3, d):
  if d == 128:
    return _make_gather128()(table, idx3)
  return _make_gather16()(table, idx3)


def _sc_gather2(table, idxa3, idxb3):
  return _make_gather16x2()(table, idxa3, idxb3)


def _sc_scatter(msg3, dst3, zeros, ones, dv, with_cnt):
  return _make_scatter(dv, with_cnt)(msg3, dst3, zeros, ones)


def _msg_body(xg_ref, ea_ref, w1_ref, b1_ref, w2p_ref, r_ref, s_ref,
              b2r_ref, o_ref):
  xg = xg_ref[...].astype(jnp.bfloat16)
  h = jnp.maximum(
      jnp.dot(ea_ref[...], w1_ref[...], preferred_element_type=jnp.float32)
      + b1_ref[...], 0.0)
  v = jnp.dot(xg, w2p_ref[...], preferred_element_type=jnp.float32)
  he = jnp.dot(h.astype(jnp.bfloat16), r_ref[...],
               preferred_element_type=jnp.float32)
  o_ref[...] = (
      jnp.dot((v * he).astype(jnp.bfloat16), s_ref[...],
              preferred_element_type=jnp.float32)
      + jnp.dot(xg, b2r_ref[...], preferred_element_type=jnp.float32))


def _run_msg(xg, ea_p, w1, b1, w2p, r_mat, s_mat, b2r, outd, ind):
  w1 = w1.astype(jnp.bfloat16)
  w2p = w2p.astype(jnp.bfloat16)
  b2r = b2r.astype(jnp.bfloat16)
  r_mat = r_mat.astype(jnp.bfloat16)
  s_mat = s_mat.astype(jnp.bfloat16)
  return pl.pallas_call(
      _msg_body,
      grid=(GEP,),
      in_specs=[
          pl.BlockSpec((BE, ind), lambda i: (i, 0)),
          pl.BlockSpec((BE, 16), lambda i: (i, 0)),
          pl.BlockSpec(w1.shape, lambda i: (0, 0)),
          pl.BlockSpec(b1.shape, lambda i: (0, 0)),
          pl.BlockSpec(w2p.shape, lambda i: (0, 0)),
          pl.BlockSpec(r_mat.shape, lambda i: (0, 0)),
          pl.BlockSpec(s_mat.shape, lambda i: (0, 0)),
          pl.BlockSpec(b2r.shape, lambda i: (0, 0)),
      ],
      out_specs=pl.BlockSpec((BE, outd), lambda i: (i, 0)),
      out_shape=jax.ShapeDtypeStruct((EP, outd), jnp.float32),
  )(xg, ea_p, w1, b1, w2p, r_mat, s_mat, b2r)


def _msg_body_packed(xgp_ref, eap_ref, w18_ref, b18_ref, w8_ref, r8_ref,
                     s8_ref, b8_ref, o_ref):
  xgp = xgp_ref[...].astype(jnp.bfloat16)
  h8 = jnp.maximum(
      jnp.dot(eap_ref[...].astype(jnp.bfloat16), w18_ref[...],
              preferred_element_type=jnp.float32) + b18_ref[...], 0.0)
  v8 = jnp.dot(xgp, w8_ref[...], preferred_element_type=jnp.float32)
  he8 = jnp.dot(h8.astype(jnp.bfloat16), r8_ref[...],
                preferred_element_type=jnp.float32)
  o_ref[...] = (
      jnp.dot((v8 * he8).astype(jnp.bfloat16), s8_ref[...],
              preferred_element_type=jnp.float32)
      + jnp.dot(xgp, b8_ref[...], preferred_element_type=jnp.float32))


def _run_msg_packed(xgp, eap8, w1, b1, w2p, b2r):
  BE8 = BE // 8
  w18 = jnp.kron(jnp.eye(8, dtype=jnp.float32), w1).astype(jnp.bfloat16)
  b18 = jnp.tile(b1.reshape(1, -1), (1, 8))
  w8 = jnp.kron(jnp.eye(8, dtype=jnp.float32), w2p).astype(jnp.bfloat16)
  r8 = jnp.asarray(np.kron(np.eye(8, dtype=np.float32), _R32)
                   ).astype(jnp.bfloat16)
  s8 = jnp.asarray(np.kron(np.eye(8, dtype=np.float32), _S32)
                   ).astype(jnp.bfloat16)
  b8 = jnp.kron(jnp.eye(8, dtype=jnp.float32), b2r).astype(jnp.bfloat16)
  return pl.pallas_call(
      _msg_body_packed,
      grid=(GEP,),
      in_specs=[
          pl.BlockSpec((BE8, 128), lambda i: (i, 0)),
          pl.BlockSpec((BE8, 128), lambda i: (i, 0)),
          pl.BlockSpec((128, 256), lambda i: (0, 0)),
          pl.BlockSpec((1, 256), lambda i: (0, 0)),
          pl.BlockSpec((128, 4096), lambda i: (0, 0)),
          pl.BlockSpec((256, 4096), lambda i: (0, 0)),
          pl.BlockSpec((4096, 128), lambda i: (0, 0)),
          pl.BlockSpec((128, 128), lambda i: (0, 0)),
      ],
      out_specs=pl.BlockSpec((BE8, 128), lambda i: (i, 0)),
      out_shape=jax.ShapeDtypeStruct((EP // 8, 128), jnp.float32),
  )(xgp, eap8, w18, b18, w8, r8, s8, b8)


def _node_body(aggp_ref, cntp_ref, f_ref, root_ref, bias_ref, o_ref):
  agg = aggp_ref[0] + aggp_ref[1]
  cnt = cntp_ref[0] + cntp_ref[1]
  o_ref[...] = jnp.maximum(
      agg / jnp.maximum(cnt, 1.0)
      + jnp.dot(f_ref[...], root_ref[...], preferred_element_type=jnp.float32)
      + bias_ref[...], 0.0)


def _run_node(aggp, cntp, f, root, bias, ind):
  return pl.pallas_call(
      _node_body,
      grid=(GN,),
      in_specs=[
          pl.BlockSpec((NC, BN, 16), lambda i: (0, i, 0)),
          pl.BlockSpec((NC, BN, 16), lambda i: (0, i, 0)),
          pl.BlockSpec((BN, ind), lambda i: (i, 0)),
          pl.BlockSpec((ind, 16), lambda i: (0, 0)),
          pl.BlockSpec((1, 16), lambda i: (0, 0)),
      ],
      out_specs=pl.BlockSpec((BN, 16), lambda i: (i, 0)),
      out_shape=jax.ShapeDtypeStruct((N, 16), jnp.float32),
  )(aggp, cntp, f, root, bias)


def _mulv_body(aggp_ref, h2_ref, mur_ref, lvr_ref, bmu_ref, blv_ref, eps_ref,
               mu_ref, lv_ref, z_ref):
  aggmu = aggp_ref[0, :, :16] + aggp_ref[1, :, :16]
  agglv = aggp_ref[0, :, 16:] + aggp_ref[1, :, 16:]
  mu = aggmu + jnp.dot(h2_ref[...], mur_ref[...],
                       preferred_element_type=jnp.float32) + bmu_ref[...]
  lv = agglv + jnp.dot(h2_ref[...], lvr_ref[...],
                       preferred_element_type=jnp.float32) + blv_ref[...]
  mu_ref[...] = mu
  lv_ref[...] = lv
  lvc = jnp.clip(lv, -5.0, 5.0)
  z_ref[...] = mu + eps_ref[...] * jnp.exp(0.5 * lvc)


_S8 = np.kron(np.eye(8, dtype=np.float32), np.ones((16, 1), np.float32))


def _dec_body(zs_ref, zd_ref, s8_ref, o_ref):
  s = jnp.dot(zs_ref[...] * zd_ref[...], s8_ref[...],
              preferred_element_type=jnp.float32)
  o_ref[...] = 1.0 / (1.0 + jnp.exp(-s))


def _pool_body(batch_ref, z_ref, sums_ref, cnt_ref):
  i = pl.program_id(0)
  b = batch_ref[0, 0, :]
  oh = (b[:, None] == lax.broadcasted_iota(jnp.int32, (BN, 64), 1)
        ).astype(jnp.float32)
  ps = lax.dot_general(oh, z_ref[...], (((0,), (0,)), ((), ())),
                       preferred_element_type=jnp.float32)
  pc = lax.dot_general(oh, jnp.ones((BN, 16), jnp.float32),
                       (((0,), (0,)), ((), ())),
                       preferred_element_type=jnp.float32)

  @pl.when(i == 0)
  def _():
    sums_ref[...] = jnp.zeros_like(sums_ref)
    cnt_ref[...] = jnp.zeros_like(cnt_ref)

  sums_ref[...] += ps
  cnt_ref[...] += pc


def _cls_body(sums_ref, cnt_ref, w1_ref, b1_ref, w2_ref, b2_ref, o_ref):
  gemb = sums_ref[...] / jnp.maximum(cnt_ref[...], 1.0)
  a = jnp.maximum(
      jnp.dot(gemb, w1_ref[...], preferred_element_type=jnp.float32)
      + b1_ref[...], 0.0)
  o_ref[...] = jnp.dot(a, w2_ref[...], preferred_element_type=jnp.float32) \
      + b2_ref[...]


def _prep_w2(w2, b2, ind, outd):
  w2p = w2.reshape(32, ind, outd).transpose(1, 0, 2).reshape(ind, 32 * outd)
  return w2p, b2.reshape(ind, outd)


_S32 = np.kron(np.ones((32, 1), np.float32), np.eye(16, dtype=np.float32))
_S64 = np.kron(np.eye(2, dtype=np.float32), _S32)


_R32 = np.kron(np.eye(32, dtype=np.float32), np.ones((1, 16), np.float32))
_R64 = np.kron(np.eye(64, dtype=np.float32), np.ones((1, 16), np.float32))


def _pad_edges(a, fill_shape_tail, fill):
  a3 = a.reshape((NW, EPW) + fill_shape_tail)
  pad = jnp.full((NW, PADW) + fill_shape_tail, fill, a.dtype)
  return jnp.concatenate([a3, pad], axis=1)


def kernel(x, edge_index, edge_attr, batch, c1_w1, c1_b1, c1_w2, c1_b2,
           c1_root, c1_bias, c2_w1, c2_b1, c2_w2, c2_b2, c2_root, c2_bias,
           cmu_w1, cmu_b1, cmu_w2, cmu_b2, cmu_root, cmu_bias, clv_w1,
           clv_b1, clv_w2, clv_b2, clv_root, clv_bias, cls_w1, cls_b1,
           cls_w2, cls_b2):
  src3 = _pad_edges(edge_index[0], (), 0).reshape(NW, NCH, CH)
  dst3 = _pad_edges(edge_index[1], (), N).reshape(NW, NCH, CH)
  ea_p = _pad_edges(edge_attr, (16,), 0).reshape(EP, 16).astype(jnp.bfloat16)
  ea_flat = jnp.concatenate(
      [edge_attr.reshape(NW, EPW * 16),
       jnp.zeros((NW, PADW * 16), jnp.float32)], axis=1)
  eap8 = ea_flat.reshape(EP // 8, 128)
  zeros16 = jnp.zeros((N, 16), jnp.float32)
  zeros32 = jnp.zeros((N, 32), jnp.float32)
  ones_rows = jnp.ones((CH, 16), jnp.float32)

  w2p1, b2r1 = _prep_w2(c1_w2, c1_b2, 128, 16)
  xg1 = _sc_gather(x, src3, 128).reshape(EP, 128)
  msg1 = _run_msg(xg1, ea_p, c1_w1, c1_b1.reshape(1, 32), w2p1,
                  jnp.asarray(_R32), jnp.asarray(_S32), b2r1, 16, 128)
  agg1p, cnt1p = _sc_scatter(msg1.reshape(NW, EPP, 16), dst3, zeros16,
                             ones_rows, 16, True)
  h1 = _run_node(agg1p, cnt1p, x, c1_root, c1_bias.reshape(1, 16), 128)

  w2p2, b2r2 = _prep_w2(c2_w2, c2_b2, 16, 16)
  xg2p = _sc_gather(h1, src3, 16).reshape(EP // 8, 128)
  msg2 = _run_msg_packed(xg2p, eap8, c2_w1, c2_b1, w2p2, b2r2)
  agg2p = _sc_scatter(msg2.reshape(NW, EPP, 16), dst3, zeros16,
                      ones_rows, 16, False)
  h2 = _run_node(agg2p, cnt1p, h1, c2_root, c2_bias.reshape(1, 16), 16)

  w2pm, b2rm = _prep_w2(cmu_w2, cmu_b2, 16, 16)
  w2pl, b2rl = _prep_w2(clv_w2, clv_b2, 16, 16)
  w2pc = jnp.concatenate([w2pm, w2pl], axis=1)
  b2rc = jnp.concatenate([b2rm, b2rl], axis=1)
  w1ml = jnp.concatenate([cmu_w1, clv_w1], axis=1)
  b1ml = jnp.concatenate([cmu_b1, clv_b1]).reshape(1, 64)
  xg3 = _sc_gather(h2, src3, 16).reshape(EP, 16)
  msg3 = _run_msg(xg3, ea_p, w1ml, b1ml, w2pc,
                  jnp.asarray(_R64), jnp.asarray(_S64), b2rc, 32, 16)
  agg3p = _sc_scatter(msg3.reshape(NW, EPP, 32), dst3, zeros32,
                      ones_rows, 32, False)

  eps = jax.random.normal(jax.random.key(42), (N, 16), dtype=jnp.float32)
  mu, logvar, z = pl.pallas_call(
      _mulv_body,
      grid=(GN,),
      in_specs=[
          pl.BlockSpec((NC, BN, 32), lambda i: (0, i, 0)),
          pl.BlockSpec((BN, 16), lambda i: (i, 0)),
          pl.BlockSpec((16, 16), lambda i: (0, 0)),
          pl.BlockSpec((16, 16), lambda i: (0, 0)),
          pl.BlockSpec((1, 16), lambda i: (0, 0)),
          pl.BlockSpec((1, 16), lambda i: (0, 0)),
          pl.BlockSpec((BN, 16), lambda i: (i, 0)),
      ],
      out_specs=[
          pl.BlockSpec((BN, 16), lambda i: (i, 0)),
          pl.BlockSpec((BN, 16), lambda i: (i, 0)),
          pl.BlockSpec((BN, 16), lambda i: (i, 0)),
      ],
      out_shape=[
          jax.ShapeDtypeStruct((N, 16), jnp.float32),
          jax.ShapeDtypeStruct((N, 16), jnp.float32),
          jax.ShapeDtypeStruct((N, 16), jnp.float32),
      ],
  )(agg3p, h2, cmu_root, clv_root, cmu_bias.reshape(1, 16),
    clv_bias.reshape(1, 16), eps)

  zs, zd = _sc_gather2(z, src3, dst3)
  BE8 = BE // 8
  adj_pred = pl.pallas_call(
      _dec_body,
      grid=(GEP,),
      in_specs=[
          pl.BlockSpec((BE8, 128), lambda i: (i, 0)),
          pl.BlockSpec((BE8, 128), lambda i: (i, 0)),
          pl.BlockSpec((128, 8), lambda i: (0, 0)),
      ],
      out_specs=pl.BlockSpec((BE8, 8), lambda i: (i, 0)),
      out_shape=jax.ShapeDtypeStruct((EP // 8, 8), jnp.float32),
  )(zs.reshape(EP // 8, 128), zd.reshape(EP // 8, 128), jnp.asarray(_S8))
  adj_pred = adj_pred.reshape(NW, EPP)[:, :EPW].reshape(E)

  sums, cnt = pl.pallas_call(
      _pool_body,
      grid=(GN,),
      in_specs=[
          pl.BlockSpec((1, 1, BN), lambda i: (i, 0, 0)),
          pl.BlockSpec((BN, 16), lambda i: (i, 0)),
      ],
      out_specs=[
          pl.BlockSpec((64, 16), lambda i: (0, 0)),
          pl.BlockSpec((64, 16), lambda i: (0, 0)),
      ],
      out_shape=[
          jax.ShapeDtypeStruct((64, 16), jnp.float32),
          jax.ShapeDtypeStruct((64, 16), jnp.float32),
      ],
  )(batch.reshape(GN, 1, BN), z)
  class_logits = pl.pallas_call(
      _cls_body,
      in_specs=[
          pl.BlockSpec((64, 16), lambda: (0, 0)),
          pl.BlockSpec((64, 16), lambda: (0, 0)),
          pl.BlockSpec((16, 64), lambda: (0, 0)),
          pl.BlockSpec((1, 64), lambda: (0, 0)),
          pl.BlockSpec((64, 6), lambda: (0, 0)),
          pl.BlockSpec((1, 6), lambda: (0, 0)),
      ],
      out_specs=pl.BlockSpec((64, 6), lambda: (0, 0)),
      out_shape=jax.ShapeDtypeStruct((64, 6), jnp.float32),
  )(sums, cnt, cls_w1, cls_b1.reshape(1, 64), cls_w2, cls_b2.reshape(1, 6))

  return (adj_pred, mu, logvar, class_logits, z)


# speedup vs baseline: 1.0658x; 1.0014x over previous
import functools

import jax
import jax.numpy as jnp
import numpy as np
from jax import lax
from jax.experimental import pallas as pl
from jax.experimental.pallas import tpu as pltpu
from jax.experimental.pallas import tpu_sc as plsc

N = 10000
E = 160000
NC = 2
NS = 16
NW = NC * NS
EPW = E // NW
CH = 128
NCH = 40
EPP = NCH * CH
PADW = EPP - EPW
EP = NW * EPP
RPT = 624
NTL = RPT * NS
TAIL = N - NTL

BE = 5120
GEP = EP // BE
BN = 2000
GN = N // BN


@functools.lru_cache(maxsize=None)
def _get_mesh():
  return plsc.VectorSubcoreMesh(
      core_axis_name="c", subcore_axis_name="s", num_cores=NC, num_subcores=NS)


_SC_PARAMS = dict(compiler_params=pltpu.CompilerParams(use_tc_tiling_on_sc=False))


def _wid():
  return lax.axis_index("s") * NC + lax.axis_index("c")


@functools.lru_cache(maxsize=None)
def _make_gather16():
  @functools.partial(
      pl.kernel, mesh=_get_mesh(), **_SC_PARAMS,
      out_type=jax.ShapeDtypeStruct((NW, EPP, 16), jnp.float32),
      scratch_types=[
          pltpu.VMEM((NCH, CH), jnp.int32),
          pltpu.VMEM((EPP, 16), jnp.float32),
          pltpu.SemaphoreType.DMA,
      ],
      name="sc_gather16",
  )
  def gk(table_hbm, idx_hbm, out_hbm, idx_v, rows_v, sem):
    w = _wid()
    pltpu.sync_copy(idx_hbm.at[w], idx_v)

    def fire(j, c):
      pltpu.async_copy(table_hbm.at[idx_v.at[j]],
                       rows_v.at[pl.ds(j * CH, CH)], sem)
      return c

    lax.fori_loop(0, NCH, fire, 0)

    def drain(j, c):
      pltpu.make_async_copy(table_hbm.at[idx_v.at[0]],
                            rows_v.at[pl.ds(0, CH)], sem).wait()
      return c

    lax.fori_loop(0, NCH, drain, 0)
    pltpu.sync_copy(rows_v, out_hbm.at[w])

  return gk


@functools.lru_cache(maxsize=None)
def _make_gather16x2():
  @functools.partial(
      pl.kernel, mesh=_get_mesh(), **_SC_PARAMS,
      out_type=(jax.ShapeDtypeStruct((NW, EPP, 16), jnp.float32),
                jax.ShapeDtypeStruct((NW, EPP, 16), jnp.float32)),
      scratch_types=[
          pltpu.VMEM((NCH, CH), jnp.int32),
          pltpu.VMEM((NCH, CH), jnp.int32),
          pltpu.VMEM((EPP, 16), jnp.float32),
          pltpu.SemaphoreType.DMA,
      ],
      name="sc_gather16x2",
  )
  def gk(table_hbm, idxa_hbm, idxb_hbm, outa_hbm, outb_hbm,
         idxa_v, idxb_v, rows_v, sem):
    w = _wid()
    pltpu.sync_copy(idxa_hbm.at[w], idxa_v)
    pltpu.sync_copy(idxb_hbm.at[w], idxb_v)
    for idx_v, out_hbm in ((idxa_v, outa_hbm), (idxb_v, outb_hbm)):

      def fire(j, c, _iv=idx_v):
        pltpu.async_copy(table_hbm.at[_iv.at[j]],
                         rows_v.at[pl.ds(j * CH, CH)], sem)
        return c

      lax.fori_loop(0, NCH, fire, 0)

      def drain(j, c, _iv=idx_v):
        pltpu.make_async_copy(table_hbm.at[_iv.at[0]],
                              rows_v.at[pl.ds(0, CH)], sem).wait()
        return c

      lax.fori_loop(0, NCH, drain, 0)
      pltpu.sync_copy(rows_v, out_hbm.at[w])

  return gk


@functools.lru_cache(maxsize=None)
def _make_gather128():
  NB = 5
  NG = NCH // NB - 1

  @functools.partial(
      pl.kernel, mesh=_get_mesh(), **_SC_PARAMS,
      out_type=jax.ShapeDtypeStruct((NW, EPP, 128), jnp.float32),
      scratch_types=[
          pltpu.VMEM((NCH, CH), jnp.int32),
          pltpu.VMEM((CH, 128), jnp.float32),
          pltpu.VMEM((CH, 128), jnp.float32),
          pltpu.VMEM((CH, 128), jnp.float32),
          pltpu.VMEM((CH, 128), jnp.float32),
          pltpu.VMEM((CH, 128), jnp.float32),
          pltpu.SemaphoreType.DMA,
          pltpu.SemaphoreType.DMA,
          pltpu.SemaphoreType.DMA,
          pltpu.SemaphoreType.DMA,
          pltpu.SemaphoreType.DMA,
      ],
      name="sc_gather128",
  )
  def gk(table_hbm, idx_hbm, out_hbm, idx_v, r0, r1, r2, r3, r4,
         s0, s1, s2, s3, s4):
    w = _wid()
    bufs = (r0, r1, r2, r3, r4)
    sems = (s0, s1, s2, s3, s4)
    pltpu.sync_copy(idx_hbm.at[w], idx_v)

    for b in range(NB):
      pltpu.async_copy(table_hbm.at[idx_v.at[b]], bufs[b], sems[b])

    def group(g, c):
      for b in range(NB):
        j = g * NB + b
        pltpu.make_async_copy(table_hbm.at[idx_v.at[0]], bufs[b],
                              sems[b]).wait()
        pltpu.sync_copy(bufs[b], out_hbm.at[w, pl.ds(j * CH, CH)])
        pltpu.async_copy(table_hbm.at[idx_v.at[j + NB]], bufs[b], sems[b])
      return c

    lax.fori_loop(0, NG, group, 0)

    for b in range(NB):
      j = NG * NB + b
      pltpu.make_async_copy(table_hbm.at[idx_v.at[0]], bufs[b], sems[b]).wait()
      pltpu.sync_copy(bufs[b], out_hbm.at[w, pl.ds(j * CH, CH)])

  return gk


@functools.lru_cache(maxsize=None)
def _make_scatter(DV, with_cnt):
  STG = EPP if DV == 16 else EPP // 2
  NPH = EPP // STG
  CPP = NCH // NPH
  out_type = [jax.ShapeDtypeStruct((NC, N, DV), jnp.float32)]
  scratch = [
      pltpu.VMEM((NCH, CH), jnp.int32),
      pltpu.VMEM((STG, DV), jnp.float32),
      pltpu.VMEM_SHARED((N + 8, DV), jnp.float32),
      pltpu.SemaphoreType.DMA,
  ]
  if with_cnt:
    out_type.append(jax.ShapeDtypeStruct((NC, N, 16), jnp.float32))
    scratch += [
        pltpu.VMEM((CH, 16), jnp.float32),
        pltpu.VMEM_SHARED((N + 8, 16), jnp.float32),
        pltpu.SemaphoreType.DMA,
    ]

  @functools.partial(
      pl.kernel, mesh=_get_mesh(),
      out_type=tuple(out_type) if with_cnt else out_type[0],
      **_SC_PARAMS,
      scratch_types=scratch, name=f"sc_scatter_{DV}_{int(with_cnt)}",
  )
  def sk(msg_hbm, dst_hbm, zeros_hbm, ones_hbm, *rest):
    if with_cnt:
      agg_out, cnt_out, idx_v, rows_v, table_sh, sem, ones_v, cnt_sh, semc = rest
    else:
      agg_out, idx_v, rows_v, table_sh, sem = rest
    cid = lax.axis_index("c")
    sid = lax.axis_index("s")
    w = sid * NC + cid
    r0 = sid * RPT

    pltpu.sync_copy(zeros_hbm.at[pl.ds(r0, RPT)], table_sh.at[pl.ds(r0, RPT)])
    if with_cnt:
      pltpu.sync_copy(zeros_hbm.at[pl.ds(r0, RPT), :16], cnt_sh.at[pl.ds(r0, RPT)])
      pltpu.sync_copy(ones_hbm, ones_v)

    @pl.when(sid == 0)
    def _zero_tail():
      pltpu.sync_copy(zeros_hbm.at[pl.ds(NTL, TAIL)], table_sh.at[pl.ds(NTL, TAIL)])
      if with_cnt:
        pltpu.sync_copy(zeros_hbm.at[pl.ds(NTL, TAIL), :16],
                        cnt_sh.at[pl.ds(NTL, TAIL)])

    pltpu.sync_copy(dst_hbm.at[w], idx_v)
    plsc.subcore_barrier()

    for p in range(NPH):
      pltpu.sync_copy(msg_hbm.at[w, pl.ds(p * STG, STG)], rows_v)

      def fire(j, c, _p=p):
        pltpu.async_copy(rows_v.at[pl.ds(j * CH, CH)],
                         table_sh.at[idx_v.at[_p * CPP + j]], sem, add=True)
        if with_cnt:
          pltpu.async_copy(ones_v, cnt_sh.at[idx_v.at[_p * CPP + j]],
                           semc, add=True)
        return c

      lax.fori_loop(0, CPP, fire, 0)

      def drain(j, c):
        pltpu.make_async_copy(msg_hbm.at[w, pl.ds(0, CH)],
                              rows_v.at[pl.ds(0, CH)], sem).wait()
        if with_cnt:
          pltpu.make_async_copy(msg_hbm.at[w, pl.ds(0, CH), :16],
                                ones_v, semc).wait()
        return c

      lax.fori_loop(0, CPP, drain, 0)

    plsc.subcore_barrier()

    pltpu.sync_copy(table_sh.at[pl.ds(r0, RPT)], agg_out.at[cid, pl.ds(r0, RPT)])
    if with_cnt:
      pltpu.sync_copy(cnt_sh.at[pl.ds(r0, RPT)], cnt_out.at[cid, pl.ds(r0, RPT)])

    @pl.when(sid == 0)
    def _write_tail():
      pltpu.sync_copy(table_sh.at[pl.ds(NTL, TAIL)],
                      agg_out.at[cid, pl.ds(NTL, TAIL)])
      if with_cnt:
        pltpu.sync_copy(cnt_sh.at[pl.ds(NTL, TAIL)],
                        cnt_out.at[cid, pl.ds(NTL, TAIL)])

  return sk


def _sc_gather(table, idx3, d):
  if d == 128:
    return _make_gather128()(table, idx3)
  return _make_gather16()(table, idx3)


def _sc_gather2(table, idxa3, idxb3):
  return _make_gather16x2()(table, idxa3, idxb3)


def _sc_scatter(msg3, dst3, zeros, ones, dv, with_cnt):
  return _make_scatter(dv, with_cnt)(msg3, dst3, zeros, ones)


def _msg_body(xg_ref, ea_ref, w1_ref, b1_ref, w2p_ref, r_ref, s_ref,
              b2r_ref, o_ref):
  xg = xg_ref[...].astype(jnp.bfloat16)
  h = jnp.maximum(
      jnp.dot(ea_ref[...], w1_ref[...], preferred_element_type=jnp.float32)
      + b1_ref[...], 0.0)
  v = jnp.dot(xg, w2p_ref[...], preferred_element_type=jnp.float32)
  he = jnp.dot(h.astype(jnp.bfloat16), r_ref[...],
               preferred_element_type=jnp.float32)
  o_ref[...] = (
      jnp.dot((v * he).astype(jnp.bfloat16), s_ref[...],
              preferred_element_type=jnp.float32)
      + jnp.dot(xg, b2r_ref[...], preferred_element_type=jnp.float32))


def _run_msg(xg, ea_p, w1, b1, w2p, r_mat, s_mat, b2r, outd, ind):
  w1 = w1.astype(jnp.bfloat16)
  w2p = w2p.astype(jnp.bfloat16)
  b2r = b2r.astype(jnp.bfloat16)
  r_mat = r_mat.astype(jnp.bfloat16)
  s_mat = s_mat.astype(jnp.bfloat16)
  return pl.pallas_call(
      _msg_body,
      grid=(GEP,),
      in_specs=[
          pl.BlockSpec((BE, ind), lambda i: (i, 0)),
          pl.BlockSpec((BE, 16), lambda i: (i, 0)),
          pl.BlockSpec(w1.shape, lambda i: (0, 0)),
          pl.BlockSpec(b1.shape, lambda i: (0, 0)),
          pl.BlockSpec(w2p.shape, lambda i: (0, 0)),
          pl.BlockSpec(r_mat.shape, lambda i: (0, 0)),
          pl.BlockSpec(s_mat.shape, lambda i: (0, 0)),
          pl.BlockSpec(b2r.shape, lambda i: (0, 0)),
      ],
      out_specs=pl.BlockSpec((BE, outd), lambda i: (i, 0)),
      out_shape=jax.ShapeDtypeStruct((EP, outd), jnp.float32),
  )(xg, ea_p, w1, b1, w2p, r_mat, s_mat, b2r)


def _msg_body_packed(xgp_ref, eap_ref, w18_ref, b18_ref, w8_ref, r8_ref,
                     s8_ref, b8_ref, o_ref):
  xgp = xgp_ref[...].astype(jnp.bfloat16)
  h8 = jnp.maximum(
      jnp.dot(eap_ref[...].astype(jnp.bfloat16), w18_ref[...],
              preferred_element_type=jnp.float32) + b18_ref[...], 0.0)
  v8 = jnp.dot(xgp, w8_ref[...], preferred_element_type=jnp.float32)
  he8 = jnp.dot(h8.astype(jnp.bfloat16), r8_ref[...],
                preferred_element_type=jnp.float32)
  o_ref[...] = (
      jnp.dot((v8 * he8).astype(jnp.bfloat16), s8_ref[...],
              preferred_element_type=jnp.float32)
      + jnp.dot(xgp, b8_ref[...], preferred_element_type=jnp.float32))


def _run_msg_packed(xgp, eap8, w1, b1, w2p, b2r):
  BE8 = BE // 8
  w18 = jnp.kron(jnp.eye(8, dtype=jnp.float32), w1).astype(jnp.bfloat16)
  b18 = jnp.tile(b1.reshape(1, -1), (1, 8))
  w8 = jnp.kron(jnp.eye(8, dtype=jnp.float32), w2p).astype(jnp.bfloat16)
  r8 = jnp.asarray(np.kron(np.eye(8, dtype=np.float32), _R32)
                   ).astype(jnp.bfloat16)
  s8 = jnp.asarray(np.kron(np.eye(8, dtype=np.float32), _S32)
                   ).astype(jnp.bfloat16)
  b8 = jnp.kron(jnp.eye(8, dtype=jnp.float32), b2r).astype(jnp.bfloat16)
  return pl.pallas_call(
      _msg_body_packed,
      grid=(GEP,),
      in_specs=[
          pl.BlockSpec((BE8, 128), lambda i: (i, 0)),
          pl.BlockSpec((BE8, 128), lambda i: (i, 0)),
          pl.BlockSpec((128, 256), lambda i: (0, 0)),
          pl.BlockSpec((1, 256), lambda i: (0, 0)),
          pl.BlockSpec((128, 4096), lambda i: (0, 0)),
          pl.BlockSpec((256, 4096), lambda i: (0, 0)),
          pl.BlockSpec((4096, 128), lambda i: (0, 0)),
          pl.BlockSpec((128, 128), lambda i: (0, 0)),
      ],
      out_specs=pl.BlockSpec((BE8, 128), lambda i: (i, 0)),
      out_shape=jax.ShapeDtypeStruct((EP // 8, 128), jnp.float32),
  )(xgp, eap8, w18, b18, w8, r8, s8, b8)


def _node_body(aggp_ref, cntp_ref, f_ref, root_ref, bias_ref, o_ref):
  agg = aggp_ref[0] + aggp_ref[1]
  cnt = cntp_ref[0] + cntp_ref[1]
  o_ref[...] = jnp.maximum(
      agg / jnp.maximum(cnt, 1.0)
      + jnp.dot(f_ref[...], root_ref[...], preferred_element_type=jnp.float32)
      + bias_ref[...], 0.0)


def _run_node(aggp, cntp, f, root, bias, ind):
  return pl.pallas_call(
      _node_body,
      grid=(GN,),
      in_specs=[
          pl.BlockSpec((NC, BN, 16), lambda i: (0, i, 0)),
          pl.BlockSpec((NC, BN, 16), lambda i: (0, i, 0)),
          pl.BlockSpec((BN, ind), lambda i: (i, 0)),
          pl.BlockSpec((ind, 16), lambda i: (0, 0)),
          pl.BlockSpec((1, 16), lambda i: (0, 0)),
      ],
      out_specs=pl.BlockSpec((BN, 16), lambda i: (i, 0)),
      out_shape=jax.ShapeDtypeStruct((N, 16), jnp.float32),
  )(aggp, cntp, f, root, bias)


def _mulv_body(aggp_ref, h2_ref, mur_ref, lvr_ref, bmu_ref, blv_ref, eps_ref,
               mu_ref, lv_ref, z_ref):
  aggmu = aggp_ref[0, :, :16] + aggp_ref[1, :, :16]
  agglv = aggp_ref[0, :, 16:] + aggp_ref[1, :, 16:]
  mu = aggmu + jnp.dot(h2_ref[...], mur_ref[...],
                       preferred_element_type=jnp.float32) + bmu_ref[...]
  lv = agglv + jnp.dot(h2_ref[...], lvr_ref[...],
                       preferred_element_type=jnp.float32) + blv_ref[...]
  mu_ref[...] = mu
  lv_ref[...] = lv
  lvc = jnp.clip(lv, -5.0, 5.0)
  z_ref[...] = mu + eps_ref[...] * jnp.exp(0.5 * lvc)


_S8 = np.kron(np.eye(8, dtype=np.float32), np.ones((16, 1), np.float32))


def _dec_body(zs_ref, zd_ref, s8_ref, o_ref):
  s = jnp.dot(zs_ref[...] * zd_ref[...], s8_ref[...],
              preferred_element_type=jnp.float32)
  o_ref[...] = 1.0 / (1.0 + jnp.exp(-s))


def _pool_body(batch_ref, z_ref, sums_ref, cnt_ref):
  i = pl.program_id(0)
  b = batch_ref[0, 0, :]
  oh = (b[:, None] == lax.broadcasted_iota(jnp.int32, (BN, 64), 1)
        ).astype(jnp.float32)
  ps = lax.dot_general(oh, z_ref[...], (((0,), (0,)), ((), ())),
                       preferred_element_type=jnp.float32)
  pc = lax.dot_general(oh, jnp.ones((BN, 16), jnp.float32),
                       (((0,), (0,)), ((), ())),
                       preferred_element_type=jnp.float32)

  @pl.when(i == 0)
  def _():
    sums_ref[...] = jnp.zeros_like(sums_ref)
    cnt_ref[...] = jnp.zeros_like(cnt_ref)

  sums_ref[...] += ps
  cnt_ref[...] += pc


def _cls_body(sums_ref, cnt_ref, w1_ref, b1_ref, w2_ref, b2_ref, o_ref):
  gemb = sums_ref[...] / jnp.maximum(cnt_ref[...], 1.0)
  a = jnp.maximum(
      jnp.dot(gemb, w1_ref[...], preferred_element_type=jnp.float32)
      + b1_ref[...], 0.0)
  o_ref[...] = jnp.dot(a, w2_ref[...], preferred_element_type=jnp.float32) \
      + b2_ref[...]


def _prep_w2(w2, b2, ind, outd):
  w2p = w2.reshape(32, ind, outd).transpose(1, 0, 2).reshape(ind, 32 * outd)
  return w2p, b2.reshape(ind, outd)


_S32 = np.kron(np.ones((32, 1), np.float32), np.eye(16, dtype=np.float32))
_S64 = np.kron(np.eye(2, dtype=np.float32), _S32)


_R32 = np.kron(np.eye(32, dtype=np.float32), np.ones((1, 16), np.float32))
_R64 = np.kron(np.eye(64, dtype=np.float32), np.ones((1, 16), np.float32))


def _pad_edges(a, fill_shape_tail, fill):
  a3 = a.reshape((NW, EPW) + fill_shape_tail)
  pad = jnp.full((NW, PADW) + fill_shape_tail, fill, a.dtype)
  return jnp.concatenate([a3, pad], axis=1)


def kernel(x, edge_index, edge_attr, batch, c1_w1, c1_b1, c1_w2, c1_b2,
           c1_root, c1_bias, c2_w1, c2_b1, c2_w2, c2_b2, c2_root, c2_bias,
           cmu_w1, cmu_b1, cmu_w2, cmu_b2, cmu_root, cmu_bias, clv_w1,
           clv_b1, clv_w2, clv_b2, clv_root, clv_bias, cls_w1, cls_b1,
           cls_w2, cls_b2):
  src3 = _pad_edges(edge_index[0], (), 0).reshape(NW, NCH, CH)
  dst3 = _pad_edges(edge_index[1], (), N).reshape(NW, NCH, CH)
  ea_p = _pad_edges(edge_attr, (16,), 0).reshape(EP, 16).astype(jnp.bfloat16)
  ea_flat = jnp.concatenate(
      [edge_attr.reshape(NW, EPW * 16),
       jnp.zeros((NW, PADW * 16), jnp.float32)], axis=1)
  eap8 = ea_flat.reshape(EP // 8, 128)
  zeros16 = jnp.zeros((N, 16), jnp.float32)
  zeros32 = jnp.zeros((N, 32), jnp.float32)
  ones_rows = jnp.ones((CH, 16), jnp.float32)

  w2p1, b2r1 = _prep_w2(c1_w2, c1_b2, 128, 16)
  xg1 = _sc_gather(x, src3, 128).reshape(EP, 128)
  msg1 = _run_msg(xg1, ea_p, c1_w1, c1_b1.reshape(1, 32), w2p1,
                  jnp.asarray(_R32), jnp.asarray(_S32), b2r1, 16, 128)
  agg1p, cnt1p = _sc_scatter(msg1.reshape(NW, EPP, 16), dst3, zeros16,
                             ones_rows, 16, True)
  h1 = _run_node(agg1p, cnt1p, x, c1_root, c1_bias.reshape(1, 16), 128)

  w2p2, b2r2 = _prep_w2(c2_w2, c2_b2, 16, 16)
  xg2p = _sc_gather(h1, src3, 16).reshape(EP // 8, 128)
  msg2 = _run_msg_packed(xg2p, eap8, c2_w1, c2_b1, w2p2, b2r2)
  agg2p = _sc_scatter(msg2.reshape(NW, EPP, 16), dst3, zeros16,
                      ones_rows, 16, False)
  h2 = _run_node(agg2p, cnt1p, h1, c2_root, c2_bias.reshape(1, 16), 16)

  w2pm, b2rm = _prep_w2(cmu_w2, cmu_b2, 16, 16)
  w2pl, b2rl = _prep_w2(clv_w2, clv_b2, 16, 16)
  w2pc = jnp.concatenate([w2pm, w2pl], axis=1)
  b2rc = jnp.concatenate([b2rm, b2rl], axis=1)
  w1ml = jnp.concatenate([cmu_w1, clv_w1], axis=1)
  b1ml = jnp.concatenate([cmu_b1, clv_b1]).reshape(1, 64)
  xg3 = _sc_gather(h2, src3, 16).reshape(EP, 16)
  msg3 = _run_msg(xg3, ea_p, w1ml, b1ml, w2pc,
                  jnp.asarray(_R64), jnp.asarray(_S64), b2rc, 32, 16)
  agg3p = _sc_scatter(msg3.reshape(NW, EPP, 32), dst3, zeros32,
                      ones_rows, 32, False)

  eps = jax.random.normal(jax.random.key(42), (N, 16), dtype=jnp.float32)
  mu, logvar, z = pl.pallas_call(
      _mulv_body,
      grid=(GN,),
      in_specs=[
          pl.BlockSpec((NC, BN, 32), lambda i: (0, i, 0)),
          pl.BlockSpec((BN, 16), lambda i: (i, 0)),
          pl.BlockSpec((16, 16), lambda i: (0, 0)),
          pl.BlockSpec((16, 16), lambda i: (0, 0)),
          pl.BlockSpec((1, 16), lambda i: (0, 0)),
          pl.BlockSpec((1, 16), lambda i: (0, 0)),
          pl.BlockSpec((BN, 16), lambda i: (i, 0)),
      ],
      out_specs=[
          pl.BlockSpec((BN, 16), lambda i: (i, 0)),
          pl.BlockSpec((BN, 16), lambda i: (i, 0)),
          pl.BlockSpec((BN, 16), lambda i: (i, 0)),
      ],
      out_shape=[
          jax.ShapeDtypeStruct((N, 16), jnp.float32),
          jax.ShapeDtypeStruct((N, 16), jnp.float32),
          jax.ShapeDtypeStruct((N, 16), jnp.float32),
      ],
  )(agg3p, h2, cmu_root, clv_root, cmu_bias.reshape(1, 16),
    clv_bias.reshape(1, 16), eps)

  zs, zd = _sc_gather2(z, src3, dst3)
  BE8 = BE // 8
  adj_pred = pl.pallas_call(
      _dec_body,
      grid=(GEP,),
      in_specs=[
          pl.BlockSpec((BE8, 128), lambda i: (i, 0)),
          pl.BlockSpec((BE8, 128), lambda i: (i, 0)),
          pl.BlockSpec((128, 8), lambda i: (0, 0)),
      ],
      out_specs=pl.BlockSpec((BE8, 8), lambda i: (i, 0)),
      out_shape=jax.ShapeDtypeStruct((EP // 8, 8), jnp.float32),
  )(zs.reshape(EP // 8, 128), zd.reshape(EP // 8, 128), jnp.asarray(_S8))
  adj_pred = adj_pred.reshape(NW, EPP)[:, :EPW].reshape(E)

  sums, cnt = pl.pallas_call(
      _pool_body,
      grid=(GN,),
      in_specs=[
          pl.BlockSpec((1, 1, BN), lambda i: (i, 0, 0)),
          pl.BlockSpec((BN, 16), lambda i: (i, 0)),
      ],
      out_specs=[
          pl.BlockSpec((64, 16), lambda i: (0, 0)),
          pl.BlockSpec((64, 16), lambda i: (0, 0)),
      ],
      out_shape=[
          jax.ShapeDtypeStruct((64, 16), jnp.float32),
          jax.ShapeDtypeStruct((64, 16), jnp.float32),
      ],
  )(batch.reshape(GN, 1, BN), z)
  class_logits = pl.pallas_call(
      _cls_body,
      in_specs=[
          pl.BlockSpec((64, 16), lambda: (0, 0)),
          pl.BlockSpec((64, 16), lambda: (0, 0)),
          pl.BlockSpec((16, 64), lambda: (0, 0)),
          pl.BlockSpec((1, 64), lambda: (0, 0)),
          pl.BlockSpec((64, 6), lambda: (0, 0)),
          pl.BlockSpec((1, 6), lambda: (0, 0)),
      ],
      out_specs=pl.BlockSpec((64, 6), lambda: (0, 0)),
      out_shape=jax.ShapeDtypeStruct((64, 6), jnp.float32),
  )(sums, cnt, cls_w1, cls_b1.reshape(1, 64), cls_w2, cls_b2.reshape(1, 6))

  return (adj_pred, mu, logvar, class_logits, z)
